# baseline (speedup 1.0000x reference)
import jax
import jax.numpy as jnp
from jax import lax
from jax.experimental import pallas as pl
from jax.experimental.pallas import tpu as pltpu
from jax.experimental.pallas import tpu_sc as plsc

VOCAB = 100000
MAXLEN = 200
EMBED = 128
BATCH = 1024

NC = 2
NS = 16
NW = NC * NS

ROWS = BATCH * MAXLEN
CHUNK = 128
NCHUNK = ROWS // CHUNK
CPW = NCHUNK // NW
NLANE = 16
EV = EMBED // NLANE
NBUF = 3
LOOPHI = ((CPW + NBUF - 1) // NBUF) * NBUF


def _body(x_hbm, tok_hbm, pos_hbm, out_hbm, pos_v, idx_all, sp, *rest):
  buf = rest[0:NBUF]
  gsem = rest[NBUF:2 * NBUF]
  xsem = rest[2 * NBUF]
  ssem = rest[2 * NBUF + 1]

  cid = lax.axis_index("c")
  sid = lax.axis_index("s")
  wid = sid * NC + cid
  wchunk0 = wid * CPW

  pltpu.sync_copy(x_hbm.at[pl.ds(wchunk0 * CHUNK, CPW * CHUNK)], idx_all)
  pltpu.sync_copy(pos_hbm, pos_v)

  def start_gather(c, b):
    pltpu.async_copy(tok_hbm.at[idx_all.at[pl.ds(c * CHUNK, CHUNK)]],
                     buf[b], gsem[b])

  for b in range(2):
    start_gather(jnp.int32(b), b)

  @pl.loop(jnp.int32(0), jnp.int32(LOOPHI), step=jnp.int32(NBUF))
  def _(g):
    for b in range(NBUF):
      c = g + b
      rowbase = (wchunk0 + c) * CHUNK
      t = lax.rem(c, jnp.int32(2))
      ot = 1 - t
      pb = (b - 1) % NBUF

      @pl.when(c < CPW)
      def _():
        pltpu.make_async_copy(
            tok_hbm.at[idx_all.at[pl.ds(c * CHUNK, CHUNK)]], buf[b],
            gsem[b]).wait()

        @pl.when(c >= 2)
        def _():
          pltpu.make_async_copy(
              sp.at[sid * 2 + t],
              out_hbm.at[pl.ds(rowbase - 2 * CHUNK, CHUNK)], ssem).wait()

        l0 = lax.rem(rowbase, jnp.int32(MAXLEN))
        @plsc.parallel_loop(jnp.int32(0), jnp.int32(CHUNK), unroll=2)
        def _(i):
          lraw = l0 + i
          l = lraw - lax.select(lraw >= MAXLEN, jnp.int32(MAXLEN),
                                jnp.int32(0))
          for j in range(EV):
            sl = pl.ds(j * NLANE, NLANE)
            plsc.addupdate(buf[b].at[i, sl], pos_v[l, sl])

        pltpu.async_copy(buf[b], sp.at[sid * 2 + t], xsem)

        @pl.when(c >= 1)
        def _():
          pltpu.make_async_copy(buf[pb], sp.at[sid * 2 + ot], xsem).wait()
          pltpu.async_copy(
              sp.at[sid * 2 + ot],
              out_hbm.at[pl.ds(rowbase - CHUNK, CHUNK)], ssem)

        @pl.when(c + 2 < CPW)
        def _():
          start_gather(c + 2, (b + 2) % NBUF)

  lastb = (CPW - 1) % NBUF
  lastt = (CPW - 1) % 2
  rowlast = (wchunk0 + CPW - 1) * CHUNK
  pltpu.make_async_copy(buf[lastb], sp.at[sid * 2 + lastt], xsem).wait()
  pltpu.async_copy(
      sp.at[sid * 2 + lastt], out_hbm.at[pl.ds(rowlast, CHUNK)], ssem)
  pltpu.make_async_copy(
      sp.at[sid * 2 + (1 - lastt)],
      out_hbm.at[pl.ds(rowlast - CHUNK, CHUNK)], ssem).wait()
  pltpu.make_async_copy(
      sp.at[sid * 2 + lastt], out_hbm.at[pl.ds(rowlast, CHUNK)],
      ssem).wait()


@jax.jit
def kernel(x, token_table, pos_table):
  x_flat = x.reshape(-1).astype(jnp.int32)
  mesh = plsc.VectorSubcoreMesh(
      core_axis_name="c", subcore_axis_name="s",
      num_cores=NC, num_subcores=NS)
  scratch = [
      pltpu.VMEM((MAXLEN, EMBED), jnp.float32),
      pltpu.VMEM((CPW * CHUNK,), jnp.int32),
      pltpu.VMEM_SHARED((NS * 2, CHUNK, EMBED), jnp.float32),
  ]
  scratch += [pltpu.VMEM((CHUNK, EMBED), jnp.float32)] * NBUF
  scratch += [pltpu.SemaphoreType.DMA] * NBUF
  scratch += [pltpu.SemaphoreType.DMA]
  scratch += [pltpu.SemaphoreType.DMA]
  f = pl.kernel(
      _body,
      out_type=jax.ShapeDtypeStruct((ROWS, EMBED), jnp.float32),
      mesh=mesh,
      scratch_types=scratch,
  )
  out = f(x_flat, token_table, pos_table)
  return out.reshape(BATCH, MAXLEN, EMBED)


# speedup vs baseline: 1.1185x; 1.1185x over previous
import jax
import jax.numpy as jnp
from jax import lax
from jax.experimental import pallas as pl
from jax.experimental.pallas import tpu as pltpu
from jax.experimental.pallas import tpu_sc as plsc

VOCAB = 100000
MAXLEN = 200
EMBED = 128
BATCH = 1024

NC = 2
NS = 16
NW = NC * NS

ROWS = BATCH * MAXLEN
CHUNK = 128
NCHUNK = ROWS // CHUNK
CPW = NCHUNK // NW
NLANE = 16
EV = EMBED // NLANE
NBUF = 4
LOOPHI = ((CPW + NBUF - 1) // NBUF) * NBUF
POSX = MAXLEN + CHUNK - 8


def _body(x_hbm, tok_hbm, pos_hbm, out_hbm, pos_v, idx_all, *rest):
  buf = rest[0:NBUF]
  gsem = rest[NBUF:2 * NBUF]
  ssem = rest[2 * NBUF:3 * NBUF]

  wid = lax.axis_index("s") * NC + lax.axis_index("c")
  wchunk0 = wid * CPW

  pltpu.sync_copy(x_hbm.at[pl.ds(wchunk0 * CHUNK, CPW * CHUNK)], idx_all)
  pltpu.sync_copy(pos_hbm, pos_v.at[pl.ds(0, MAXLEN)])
  pltpu.sync_copy(pos_hbm.at[pl.ds(0, POSX - MAXLEN)],
                  pos_v.at[pl.ds(MAXLEN, POSX - MAXLEN)])

  def start_gather(c, b):
    pltpu.async_copy(tok_hbm.at[idx_all.at[pl.ds(c * CHUNK, CHUNK)]],
                     buf[b], gsem[b])

  for b in range(2):
    start_gather(jnp.int32(b), b)

  @pl.loop(jnp.int32(0), jnp.int32(LOOPHI), step=jnp.int32(NBUF))
  def _(g):
    for b in range(NBUF):
      c = g + b
      rowbase = (wchunk0 + c) * CHUNK

      @pl.when(c < CPW)
      def _():
        pltpu.make_async_copy(
            tok_hbm.at[idx_all.at[pl.ds(c * CHUNK, CHUNK)]], buf[b],
            gsem[b]).wait()

        l0 = lax.rem(rowbase, jnp.int32(MAXLEN))
        @plsc.parallel_loop(jnp.int32(0), jnp.int32(CHUNK), unroll=4)
        def _(i):
          l = l0 + i
          for j in range(EV):
            sl = pl.ds(j * NLANE, NLANE)
            plsc.addupdate(buf[b].at[i, sl], pos_v[l, sl])

        b2 = (b + 2) % NBUF
        @pl.when(c >= 2)
        def _():
          pltpu.make_async_copy(
              buf[b2], out_hbm.at[pl.ds(rowbase - 2 * CHUNK, CHUNK)],
              ssem[b2]).wait()
        @pl.when(c + 2 < CPW)
        def _():
          start_gather(c + 2, b2)

        pltpu.async_copy(buf[b], out_hbm.at[pl.ds(rowbase, CHUNK)], ssem[b])

  for k in range(2):
    c = CPW - 2 + k
    rowbase = (wchunk0 + c) * CHUNK
    pltpu.make_async_copy(
        buf[c % NBUF], out_hbm.at[pl.ds(rowbase, CHUNK)],
        ssem[c % NBUF]).wait()


@jax.jit
def kernel(x, token_table, pos_table):
  x_flat = x.reshape(-1).astype(jnp.int32)
  mesh = plsc.VectorSubcoreMesh(
      core_axis_name="c", subcore_axis_name="s",
      num_cores=NC, num_subcores=NS)
  scratch = [
      pltpu.VMEM((POSX, EMBED), jnp.float32),
      pltpu.VMEM((CPW * CHUNK,), jnp.int32),
  ]
  scratch += [pltpu.VMEM((CHUNK, EMBED), jnp.float32)] * NBUF
  scratch += [pltpu.SemaphoreType.DMA] * (2 * NBUF)
  f = pl.kernel(
      _body,
      out_type=jax.ShapeDtypeStruct((ROWS, EMBED), jnp.float32),
      mesh=mesh,
      scratch_types=scratch,
  )
  out = f(x_flat, token_table, pos_table)
  return out.reshape(BATCH, MAXLEN, EMBED)


# speedup vs baseline: 1.1832x; 1.0579x over previous
import jax
import jax.numpy as jnp
from jax import lax
from jax.experimental import pallas as pl
from jax.experimental.pallas import tpu as pltpu
from jax.experimental.pallas import tpu_sc as plsc

VOCAB = 100000
MAXLEN = 200
EMBED = 128
BATCH = 1024

NC = 2
NS = 16
NW = NC * NS

ROWS = BATCH * MAXLEN
CHUNK = 128
NCHUNK = ROWS // CHUNK
CPW = NCHUNK // NW
NLANE = 16
EV = EMBED // NLANE
NBUF = 4
LOOPHI = ((CPW + NBUF - 1) // NBUF) * NBUF
POSX = MAXLEN + CHUNK - 8


def _body(x_hbm, tok_hbm, pos_hbm, out_hbm, pos_v, idx_all, *rest):
  buf = rest[0:NBUF]
  gsem = rest[NBUF:2 * NBUF]
  ssem = rest[2 * NBUF:3 * NBUF]

  wid = lax.axis_index("s") * NC + lax.axis_index("c")
  wchunk0 = wid * CPW

  pltpu.sync_copy(x_hbm.at[pl.ds(wchunk0 * CHUNK, CPW * CHUNK)], idx_all)
  pltpu.sync_copy(pos_hbm, pos_v.at[pl.ds(0, MAXLEN)])
  pltpu.sync_copy(pos_hbm.at[pl.ds(0, POSX - MAXLEN)],
                  pos_v.at[pl.ds(MAXLEN, POSX - MAXLEN)])

  def start_gather(c, b):
    pltpu.async_copy(tok_hbm.at[idx_all.at[pl.ds(c * CHUNK, CHUNK)]],
                     buf[b], gsem[b])

  for b in range(2):
    start_gather(jnp.int32(b), b)

  @pl.loop(jnp.int32(0), jnp.int32(LOOPHI), step=jnp.int32(NBUF))
  def _(g):
    for b in range(NBUF):
      c = g + b
      rowbase = (wchunk0 + c) * CHUNK

      @pl.when(c < CPW)
      def _():
        b2 = (b + 2) % NBUF
        @pl.when(c >= 2)
        def _():
          pltpu.make_async_copy(
              buf[b2], out_hbm.at[pl.ds(rowbase - 2 * CHUNK, CHUNK)],
              ssem[b2]).wait()
        @pl.when(c + 2 < CPW)
        def _():
          start_gather(c + 2, b2)

        pltpu.make_async_copy(
            tok_hbm.at[idx_all.at[pl.ds(c * CHUNK, CHUNK)]], buf[b],
            gsem[b]).wait()

        l0 = lax.rem(rowbase, jnp.int32(MAXLEN))
        @plsc.parallel_loop(jnp.int32(0), jnp.int32(CHUNK), unroll=2)
        def _(i):
          l = l0 + i
          for j in range(EV):
            sl = pl.ds(j * NLANE, NLANE)
            plsc.addupdate(buf[b].at[i, sl], pos_v[l, sl])

        pltpu.async_copy(buf[b], out_hbm.at[pl.ds(rowbase, CHUNK)], ssem[b])

  for k in range(2):
    c = CPW - 2 + k
    rowbase = (wchunk0 + c) * CHUNK
    pltpu.make_async_copy(
        buf[c % NBUF], out_hbm.at[pl.ds(rowbase, CHUNK)],
        ssem[c % NBUF]).wait()


@jax.jit
def kernel(x, token_table, pos_table):
  x_flat = x.reshape(-1).astype(jnp.int32)
  mesh = plsc.VectorSubcoreMesh(
      core_axis_name="c", subcore_axis_name="s",
      num_cores=NC, num_subcores=NS)
  scratch = [
      pltpu.VMEM((POSX, EMBED), jnp.float32),
      pltpu.VMEM((CPW * CHUNK,), jnp.int32),
  ]
  scratch += [pltpu.VMEM((CHUNK, EMBED), jnp.float32)] * NBUF
  scratch += [pltpu.SemaphoreType.DMA] * (2 * NBUF)
  f = pl.kernel(
      _body,
      out_type=jax.ShapeDtypeStruct((ROWS, EMBED), jnp.float32),
      mesh=mesh,
      scratch_types=scratch,
  )
  out = f(x_flat, token_table, pos_table)
  return out.reshape(BATCH, MAXLEN, EMBED)


# speedup vs baseline: 1.2081x; 1.0210x over previous
import jax
import jax.numpy as jnp
from jax import lax
from jax.experimental import pallas as pl
from jax.experimental.pallas import tpu as pltpu
from jax.experimental.pallas import tpu_sc as plsc

VOCAB = 100000
MAXLEN = 200
EMBED = 128
BATCH = 1024

NC = 2
NS = 16
NW = NC * NS

ROWS = BATCH * MAXLEN
CHUNK = 128
NCHUNK = ROWS // CHUNK
CPW = NCHUNK // NW
NLANE = 16
EV = EMBED // NLANE
NBUF = 4
LOOPHI = ((CPW + NBUF - 1) // NBUF) * NBUF
POSX = MAXLEN + CHUNK - 8


def _body(x_hbm, tok_hbm, pos_hbm, out_hbm, pos_v, idx_all, *rest):
  buf = rest[0:NBUF]
  gsem = rest[NBUF:2 * NBUF]
  ssem = rest[2 * NBUF:3 * NBUF]
  psem = rest[3 * NBUF]

  wid = lax.axis_index("s") * NC + lax.axis_index("c")
  wchunk0 = wid * CPW

  pcp0 = pltpu.async_copy(pos_hbm, pos_v.at[pl.ds(0, MAXLEN)], psem)
  pcp1 = pltpu.async_copy(pos_hbm.at[pl.ds(0, POSX - MAXLEN)],
                          pos_v.at[pl.ds(MAXLEN, POSX - MAXLEN)], psem)
  pltpu.sync_copy(x_hbm.at[pl.ds(wchunk0 * CHUNK, CPW * CHUNK)], idx_all)

  def start_gather(c, b):
    pltpu.async_copy(tok_hbm.at[idx_all.at[pl.ds(c * CHUNK, CHUNK)]],
                     buf[b], gsem[b])

  for b in range(2):
    start_gather(jnp.int32(b), b)
  pcp0.wait()
  pcp1.wait()

  @pl.loop(jnp.int32(0), jnp.int32(LOOPHI), step=jnp.int32(NBUF))
  def _(g):
    for b in range(NBUF):
      c = g + b
      rowbase = (wchunk0 + c) * CHUNK

      @pl.when(c < CPW)
      def _():
        pltpu.make_async_copy(
            tok_hbm.at[idx_all.at[pl.ds(c * CHUNK, CHUNK)]], buf[b],
            gsem[b]).wait()

        b2 = (b + 2) % NBUF
        @pl.when(c >= 2)
        def _():
          pltpu.make_async_copy(
              buf[b2], out_hbm.at[pl.ds(rowbase - 2 * CHUNK, CHUNK)],
              ssem[b2]).wait()
        @pl.when(c + 2 < CPW)
        def _():
          start_gather(c + 2, b2)

        l0 = lax.rem(rowbase, jnp.int32(MAXLEN))
        @plsc.parallel_loop(jnp.int32(0), jnp.int32(CHUNK), unroll=2)
        def _(i):
          l = l0 + i
          for j in range(EV):
            sl = pl.ds(j * NLANE, NLANE)
            plsc.addupdate(buf[b].at[i, sl], pos_v[l, sl])

        pltpu.async_copy(buf[b], out_hbm.at[pl.ds(rowbase, CHUNK)], ssem[b])

  for k in range(2):
    c = CPW - 2 + k
    rowbase = (wchunk0 + c) * CHUNK
    pltpu.make_async_copy(
        buf[c % NBUF], out_hbm.at[pl.ds(rowbase, CHUNK)],
        ssem[c % NBUF]).wait()


@jax.jit
def kernel(x, token_table, pos_table):
  x_flat = x.reshape(-1).astype(jnp.int32)
  mesh = plsc.VectorSubcoreMesh(
      core_axis_name="c", subcore_axis_name="s",
      num_cores=NC, num_subcores=NS)
  scratch = [
      pltpu.VMEM((POSX, EMBED), jnp.float32),
      pltpu.VMEM((CPW * CHUNK,), jnp.int32),
  ]
  scratch += [pltpu.VMEM((CHUNK, EMBED), jnp.float32)] * NBUF
  scratch += [pltpu.SemaphoreType.DMA] * (2 * NBUF)
  scratch += [pltpu.SemaphoreType.DMA]
  f = pl.kernel(
      _body,
      out_type=jax.ShapeDtypeStruct((ROWS, EMBED), jnp.float32),
      mesh=mesh,
      scratch_types=scratch,
  )
  out = f(x_flat, token_table, pos_table)
  return out.reshape(BATCH, MAXLEN, EMBED)


# speedup vs baseline: 1.2104x; 1.0019x over previous
import jax
import jax.numpy as jnp
from jax import lax
from jax.experimental import pallas as pl
from jax.experimental.pallas import tpu as pltpu
from jax.experimental.pallas import tpu_sc as plsc

VOCAB = 100000
MAXLEN = 200
EMBED = 128
BATCH = 1024

NC = 2
NS = 16
NW = NC * NS

ROWS = BATCH * MAXLEN
CHUNK = 128
NCHUNK = ROWS // CHUNK
CPW = NCHUNK // NW
NLANE = 16
EV = EMBED // NLANE
NBUF = 4
LOOPHI = ((CPW + NBUF - 1) // NBUF) * NBUF
POSX = MAXLEN + CHUNK - 8


def _body(x_hbm, tok_hbm, pos_hbm, out_hbm, pos_v, idx_all, *rest):
  buf = rest[0:NBUF]
  gsem = rest[NBUF:2 * NBUF]
  ssem = rest[2 * NBUF:3 * NBUF]
  psem = rest[3 * NBUF]

  wid = lax.axis_index("s") * NC + lax.axis_index("c")
  wchunk0 = wid * CPW

  pcp0 = pltpu.async_copy(pos_hbm, pos_v.at[pl.ds(0, MAXLEN)], psem)
  pcp1 = pltpu.async_copy(pos_hbm.at[pl.ds(0, POSX - MAXLEN)],
                          pos_v.at[pl.ds(MAXLEN, POSX - MAXLEN)], psem)
  pltpu.sync_copy(x_hbm.at[pl.ds(wchunk0 * CHUNK, CPW * CHUNK)], idx_all)

  H = CHUNK // 2

  def start_gather(c, b):
    pltpu.async_copy(tok_hbm.at[idx_all.at[pl.ds(c * CHUNK, H)]],
                     buf[b].at[pl.ds(0, H)], gsem[b])
    pltpu.async_copy(tok_hbm.at[idx_all.at[pl.ds(c * CHUNK + H, H)]],
                     buf[b].at[pl.ds(H, H)], gsem[b])

  for b in range(2):
    start_gather(jnp.int32(b), b)
  pcp0.wait()
  pcp1.wait()

  @pl.loop(jnp.int32(0), jnp.int32(LOOPHI), step=jnp.int32(NBUF))
  def _(g):
    for b in range(NBUF):
      c = g + b
      rowbase = (wchunk0 + c) * CHUNK

      @pl.when(c < CPW)
      def _():
        pltpu.make_async_copy(
            tok_hbm.at[idx_all.at[pl.ds(c * CHUNK, H)]],
            buf[b].at[pl.ds(0, H)], gsem[b]).wait()
        pltpu.make_async_copy(
            tok_hbm.at[idx_all.at[pl.ds(c * CHUNK + H, H)]],
            buf[b].at[pl.ds(H, H)], gsem[b]).wait()

        b2 = (b + 2) % NBUF
        @pl.when(c >= 2)
        def _():
          pltpu.make_async_copy(
              buf[b2], out_hbm.at[pl.ds(rowbase - 2 * CHUNK, CHUNK)],
              ssem[b2]).wait()
        @pl.when(c + 2 < CPW)
        def _():
          start_gather(c + 2, b2)

        l0 = lax.rem(rowbase, jnp.int32(MAXLEN))
        @plsc.parallel_loop(jnp.int32(0), jnp.int32(CHUNK), unroll=2)
        def _(i):
          l = l0 + i
          for j in range(EV):
            sl = pl.ds(j * NLANE, NLANE)
            plsc.addupdate(buf[b].at[i, sl], pos_v[l, sl])

        pltpu.async_copy(buf[b], out_hbm.at[pl.ds(rowbase, CHUNK)], ssem[b])

  for k in range(2):
    c = CPW - 2 + k
    rowbase = (wchunk0 + c) * CHUNK
    pltpu.make_async_copy(
        buf[c % NBUF], out_hbm.at[pl.ds(rowbase, CHUNK)],
        ssem[c % NBUF]).wait()


@jax.jit
def kernel(x, token_table, pos_table):
  x_flat = x.reshape(-1).astype(jnp.int32)
  mesh = plsc.VectorSubcoreMesh(
      core_axis_name="c", subcore_axis_name="s",
      num_cores=NC, num_subcores=NS)
  scratch = [
      pltpu.VMEM((POSX, EMBED), jnp.float32),
      pltpu.VMEM((CPW * CHUNK,), jnp.int32),
  ]
  scratch += [pltpu.VMEM((CHUNK, EMBED), jnp.float32)] * NBUF
  scratch += [pltpu.SemaphoreType.DMA] * (2 * NBUF)
  scratch += [pltpu.SemaphoreType.DMA]
  f = pl.kernel(
      _body,
      out_type=jax.ShapeDtypeStruct((ROWS, EMBED), jnp.float32),
      mesh=mesh,
      scratch_types=scratch,
  )
  out = f(x_flat, token_table, pos_table)
  return out.reshape(BATCH, MAXLEN, EMBED)
